# Initial kernel scaffold; baseline (speedup 1.0000x reference)
import jax
import jax.numpy as jnp
from jax.experimental import pallas as pl


def kernel(U, V, pretrained, u_pos, v_pos, v_neg, batch_size):
    raise NotImplementedError("write your pallas kernel here")


# speedup vs baseline: 3.8217x; 3.8217x over previous
import functools

import jax
import jax.numpy as jnp
from jax import lax
from jax.experimental import pallas as pl
from jax.experimental.pallas import tpu as pltpu
from jax.experimental.pallas import tpu_sc as plsc

VOCAB = 1000000
DIM = 64
REG = 1e-06
N_NEG = 20

NC = 2
NS = 16
NW = NC * NS
L = 16

CB = 32
NEG_ROWS = CB * N_NEG
NIDX_ROWS = NEG_ROWS // 128


def _sc_gather_dot(U, V, P, u_pos, v_pos, vneg2, B):
    nb = B // NW
    nch = nb // CB
    nidx_per_w = nb * N_NEG // 128

    mesh = plsc.VectorSubcoreMesh(core_axis_name="c", subcore_axis_name="s")

    @functools.partial(
        pl.kernel,
        out_type=(
            jax.ShapeDtypeStruct((B, L), jnp.float32),
            jax.ShapeDtypeStruct((B, L), jnp.float32),
            jax.ShapeDtypeStruct((NW, L), jnp.float32),
        ),
        mesh=mesh,
        compiler_params=pltpu.CompilerParams(use_tc_tiling_on_sc=False),
        scratch_types=[
            pltpu.VMEM((nb,), jnp.int32),
            pltpu.VMEM((nb,), jnp.int32),
            pltpu.VMEM((nidx_per_w, 128), jnp.int32),
            pltpu.VMEM((CB, DIM), jnp.float32),
            pltpu.VMEM((CB, DIM), jnp.float32),
            pltpu.VMEM((CB, DIM), jnp.float32),
            pltpu.VMEM((NEG_ROWS, DIM), jnp.float32),
            pltpu.VMEM((CB, L), jnp.float32),
            pltpu.VMEM((CB, L), jnp.float32),
            pltpu.VMEM((L,), jnp.float32),
            pltpu.SemaphoreType.DMA,
        ],
    )
    def k(u_hbm, v_hbm, p_hbm, up_hbm, vp_hbm, vn_hbm,
          pos_out, neg_out, reg_out,
          uidx, vidx, nidx, urows, vrows, prows, nrows, posb, negb, regv, sem):
        wid = lax.axis_index("s") * NC + lax.axis_index("c")
        base = wid * nb
        zero = jnp.zeros((L,), jnp.float32)

        def b_body(b, racc):
            u = [urows[b, pl.ds(L * t, L)] for t in range(4)]
            v = [vrows[b, pl.ds(L * t, L)] for t in range(4)]
            pp = [prows[b, pl.ds(L * t, L)] for t in range(4)]
            posb[b, :] = u[0] * v[0] + u[1] * v[1] + u[2] * v[2] + u[3] * v[3]
            racc = (racc + jnp.abs(u[0] - pp[0]) + jnp.abs(u[1] - pp[1])
                    + jnp.abs(u[2] - pp[2]) + jnp.abs(u[3] - pp[3]))

            def n_body(n, accs):
                r = b * N_NEG + n
                return tuple(accs[t] + nrows[r, pl.ds(L * t, L)] for t in range(4))

            a = lax.fori_loop(0, N_NEG, n_body, (zero, zero, zero, zero))
            negb[b, :] = a[0] * u[0] + a[1] * u[1] + a[2] * u[2] + a[3] * u[3]
            return racc

        def c_body(ci, racc):
            b0 = base + ci * CB
            cb0 = ci * CB
            hs = [pltpu.async_copy(u_hbm.at[uidx.at[pl.ds(cb0, CB)]], urows, sem),
                  pltpu.async_copy(v_hbm.at[vidx.at[pl.ds(cb0, CB)]], vrows, sem),
                  pltpu.async_copy(p_hbm.at[uidx.at[pl.ds(cb0, CB)]], prows, sem)]
            for j in range(NIDX_ROWS):
                hs.append(pltpu.async_copy(v_hbm.at[nidx.at[ci * NIDX_ROWS + j]],
                                           nrows.at[pl.ds(j * 128, 128)], sem))
            for h in hs:
                h.wait()
            racc = lax.fori_loop(0, CB, b_body, racc)
            pltpu.sync_copy(posb, pos_out.at[pl.ds(b0, CB)])
            pltpu.sync_copy(negb, neg_out.at[pl.ds(b0, CB)])
            return racc

        pltpu.sync_copy(up_hbm.at[pl.ds(base, nb)], uidx)
        pltpu.sync_copy(vp_hbm.at[pl.ds(base, nb)], vidx)
        pltpu.sync_copy(vn_hbm.at[pl.ds(wid * nidx_per_w, nidx_per_w)], nidx)
        racc = lax.fori_loop(0, nch, c_body, zero)
        regv[...] = racc
        pltpu.sync_copy(regv, reg_out.at[wid])

    return k(U, V, P, u_pos, v_pos, vneg2)


def _tc_finalize(pos, neg, regp, B):
    def body(pos_ref, neg_ref, reg_ref, o_ref):
        s = jnp.sum(pos_ref[...], axis=1)
        t = jnp.sum(neg_ref[...], axis=1)
        ls = jnp.minimum(s, 0.0) - jnp.log1p(jnp.exp(-jnp.abs(s)))
        lt = jnp.minimum(-t, 0.0) - jnp.log1p(jnp.exp(-jnp.abs(t)))
        total = jnp.sum(ls + lt)
        reg = REG * jnp.sum(reg_ref[...])
        o_ref[...] = jnp.reshape(-(total / B) - reg, (1, 1))

    return pl.pallas_call(
        body, out_shape=jax.ShapeDtypeStruct((1, 1), jnp.float32),
    )(pos, neg, regp)


def kernel(U, V, pretrained, u_pos, v_pos, v_neg, batch_size):
    B = u_pos.shape[0]
    up = u_pos.astype(jnp.int32)
    vp = v_pos.astype(jnp.int32)
    vn2 = v_neg.astype(jnp.int32).reshape(B * N_NEG // 128, 128)
    pos, neg, regp = _sc_gather_dot(U, V, pretrained, up, vp, vn2, B)
    out = _tc_finalize(pos, neg, regp, B)
    return out[0, 0]
